# Initial kernel scaffold; baseline (speedup 1.0000x reference)
import jax
import jax.numpy as jnp
from jax.experimental import pallas as pl


def kernel(text_emb, image_emb, Wt, bt, Wi, bi, Wg, bg, W1, b1, W2, b2, noise):
    raise NotImplementedError("write your pallas kernel here")


# speedup vs baseline: 2.3783x; 2.3783x over previous
import functools

import jax
import jax.numpy as jnp
from jax.experimental import pallas as pl
from jax.experimental.pallas import tpu as pltpu

N = 8192
TD = 768
ID = 768
H = 512
OD = 768
E = 8
NOISE_STD = 1.0

TA = 512
TB = 1024


def _proj_gate_body(xt_ref, xi_ref, wt_ref, bt_ref, wi_ref, bi_ref,
                    wg_ref, bg_ref, noise_ref, comb_ref, gates_ref):
    tp = jnp.dot(xt_ref[...], wt_ref[...], preferred_element_type=jnp.float32)
    tp = tp + bt_ref[...]
    ip = jnp.dot(xi_ref[...], wi_ref[...], preferred_element_type=jnp.float32)
    ip = ip + bi_ref[...]
    comb = jnp.concatenate([tp, ip], axis=1)
    comb_ref[...] = comb

    logits = jnp.dot(comb, wg_ref[...], preferred_element_type=jnp.float32)
    logits = logits + bg_ref[...] + noise_ref[...] * NOISE_STD

    lane = jax.lax.broadcasted_iota(jnp.int32, (TA, E), 1)
    m1 = jnp.max(logits, axis=1, keepdims=True)
    is1 = logits == m1
    idx1 = jnp.min(jnp.where(is1, lane, E), axis=1, keepdims=True)
    masked = jnp.where(lane == idx1, -jnp.inf, logits)
    m2 = jnp.max(masked, axis=1, keepdims=True)
    is2 = masked == m2
    idx2 = jnp.min(jnp.where(is2, lane, E), axis=1, keepdims=True)
    z = jnp.exp(m2 - m1)
    w1 = 1.0 / (1.0 + z)
    w2 = 1.0 - w1
    gates_ref[...] = jnp.where(lane == idx1, w1,
                               jnp.where(lane == idx2, w2, 0.0))


def _moe_dense_body(comb_ref, gates_ref, w1_ref, b1_ref, w2_ref, b2_ref,
                    out_ref):
    e = pl.program_id(1)
    x = comb_ref[...]
    h = jnp.dot(x, w1_ref[0], preferred_element_type=jnp.float32)
    h = jnp.maximum(h + b1_ref[0], 0.0)
    y = jnp.dot(h, w2_ref[0], preferred_element_type=jnp.float32)
    y = y + b2_ref[0]
    lane = jax.lax.broadcasted_iota(jnp.int32, (TB, E), 1)
    g = jnp.sum(jnp.where(lane == e, gates_ref[...], 0.0), axis=1,
                keepdims=True)
    contrib = g * y

    @pl.when(e == 0)
    def _():
        out_ref[...] = contrib

    @pl.when(e > 0)
    def _():
        out_ref[...] += contrib


def kernel(text_emb, image_emb, Wt, bt, Wi, bi, Wg, bg, W1, b1, W2, b2, noise):
    grid_a = (N // TA,)
    comb, gates = pl.pallas_call(
        _proj_gate_body,
        grid=grid_a,
        in_specs=[
            pl.BlockSpec((TA, TD), lambda t: (t, 0)),
            pl.BlockSpec((TA, ID), lambda t: (t, 0)),
            pl.BlockSpec((TD, H), lambda t: (0, 0)),
            pl.BlockSpec((H,), lambda t: (0,)),
            pl.BlockSpec((ID, H), lambda t: (0, 0)),
            pl.BlockSpec((H,), lambda t: (0,)),
            pl.BlockSpec((2 * H, E), lambda t: (0, 0)),
            pl.BlockSpec((E,), lambda t: (0,)),
            pl.BlockSpec((TA, E), lambda t: (t, 0)),
        ],
        out_specs=[
            pl.BlockSpec((TA, 2 * H), lambda t: (t, 0)),
            pl.BlockSpec((TA, E), lambda t: (t, 0)),
        ],
        out_shape=[
            jax.ShapeDtypeStruct((N, 2 * H), jnp.float32),
            jax.ShapeDtypeStruct((N, E), jnp.float32),
        ],
        compiler_params=pltpu.CompilerParams(
            dimension_semantics=("arbitrary",)),
    )(text_emb, image_emb, Wt, bt, Wi, bi, Wg, bg, noise)

    out = pl.pallas_call(
        _moe_dense_body,
        grid=(N // TB, E),
        in_specs=[
            pl.BlockSpec((TB, 2 * H), lambda t, e: (t, 0)),
            pl.BlockSpec((TB, E), lambda t, e: (t, 0)),
            pl.BlockSpec((1, 2 * H, H), lambda t, e: (e, 0, 0)),
            pl.BlockSpec((1, 1, H), lambda t, e: (e, 0, 0)),
            pl.BlockSpec((1, H, OD), lambda t, e: (e, 0, 0)),
            pl.BlockSpec((1, 1, OD), lambda t, e: (e, 0, 0)),
        ],
        out_specs=pl.BlockSpec((TB, OD), lambda t, e: (t, 0)),
        out_shape=jax.ShapeDtypeStruct((N, OD), jnp.float32),
        compiler_params=pltpu.CompilerParams(
            dimension_semantics=("arbitrary", "arbitrary")),
    )(comb, gates, W1, b1[:, None, :], W2, b2[:, None, :])
    return out
